# Initial kernel scaffold; baseline (speedup 1.0000x reference)
import jax
import jax.numpy as jnp
from jax.experimental import pallas as pl


def kernel(x_user, x_item, edge_index_user_item, edge_index_item_user, W_user, b_user, W_item, b_item, Wl_ui, bl_ui, Wr_ui, Wl_iu, bl_iu, Wr_iu, ln_g_user, ln_b_user, ln_g_item, ln_b_item):
    raise NotImplementedError("write your pallas kernel here")


# speedup vs baseline: 6.0190x; 6.0190x over previous
import functools

import jax
import jax.numpy as jnp
from jax import lax
from jax.experimental import pallas as pl
from jax.experimental.pallas import tpu as pltpu
from jax.experimental.pallas import tpu_sc as plsc

N = 10000
E = 320000
D = 128
NC = 2
NS = 16
NW = NC * NS
EPT = E // NW
K = 128
NCHUNK = EPT // K
TAIL = EPT - NCHUNK * K
NPAD = 10240
RPT = NPAD // NS

ROWBLK = 1000
NB = N // ROWBLK


def _sc_segment_sums(h_user, h_item, src0, dst0, src1, dst1):
  zrows = jnp.zeros((RPT, D), jnp.float32)
  zhist = jnp.zeros((NPAD,), jnp.float32)

  mesh = plsc.VectorSubcoreMesh(core_axis_name="c", subcore_axis_name="s")

  @functools.partial(
      pl.kernel,
      out_type=(
          jax.ShapeDtypeStruct((2 * NC * NPAD, D), jnp.float32),
          jax.ShapeDtypeStruct((2 * NW * NPAD,), jnp.float32),
      ),
      mesh=mesh,
      compiler_params=pltpu.CompilerParams(needs_layout_passes=False),
      scratch_types=[
          pltpu.VMEM((K,), jnp.int32),
          pltpu.VMEM((K,), jnp.int32),
          pltpu.VMEM((TAIL,), jnp.int32),
          pltpu.VMEM((TAIL,), jnp.int32),
          pltpu.VMEM((K, D), jnp.float32),
          pltpu.VMEM((TAIL, D), jnp.float32),
          pltpu.VMEM((NPAD,), jnp.float32),
          pltpu.VMEM_SHARED((NPAD, D), jnp.float32),
          pltpu.SemaphoreType.DMA,
      ],
  )
  def seg(hu, hi, s0, d0, s1, d1, zr, zh, agg_out, cnt_out,
          idx_s, idx_d, idx_st, idx_dt, rows, rows_t, hist, acc, sem):
    c = lax.axis_index("c")
    s = lax.axis_index("s")
    wid = c * NS + s
    rowbase = pl.multiple_of(s * RPT, 8)
    ebase = wid * EPT
    ones16 = jnp.ones((16,), jnp.float32)

    def zero_owned():
      pltpu.sync_copy(zr, acc.at[pl.ds(rowbase, RPT)])
      pltpu.sync_copy(zh, hist)

    zero_owned()
    plsc.subcore_barrier()

    def do_rel(rel, table, src_hbm, dst_hbm):
      def chunk(j, carry):
        eoff = pl.multiple_of(ebase + j * K, 8)
        pltpu.sync_copy(src_hbm.at[pl.ds(eoff, K)], idx_s)
        pltpu.sync_copy(dst_hbm.at[pl.ds(eoff, K)], idx_d)
        pltpu.async_copy(table.at[idx_s], rows, sem).wait()
        pltpu.sync_copy(rows, acc.at[idx_d], add=True)
        for t in range(K // 16):
          plsc.addupdate_scatter(hist, [idx_d[pl.ds(t * 16, 16)]], ones16)
        return carry
      lax.fori_loop(0, NCHUNK, chunk, 0)
      toff = pl.multiple_of(ebase + NCHUNK * K, 8)
      pltpu.sync_copy(src_hbm.at[pl.ds(toff, TAIL)], idx_st)
      pltpu.sync_copy(dst_hbm.at[pl.ds(toff, TAIL)], idx_dt)
      pltpu.async_copy(table.at[idx_st], rows_t, sem).wait()
      pltpu.sync_copy(rows_t, acc.at[idx_dt], add=True)
      plsc.addupdate_scatter(hist, [idx_dt[...]], ones16)
      plsc.subcore_barrier()
      obase = pl.multiple_of((rel * NC + c) * NPAD + rowbase, 8)
      pltpu.sync_copy(acc.at[pl.ds(rowbase, RPT)],
                      agg_out.at[pl.ds(obase, RPT)])
      hbase = pl.multiple_of((rel * NW + wid) * NPAD, 8)
      pltpu.sync_copy(hist, cnt_out.at[pl.ds(hbase, NPAD)])

    do_rel(0, hu, s0, d0)
    zero_owned()
    plsc.subcore_barrier()
    do_rel(1, hi, s1, d1)

  agg, cnt = seg(h_user, h_item, src0, dst0, src1, dst1, zrows, zhist)
  return (agg.reshape(2, NC, NPAD, D), cnt.reshape(2, NW, NPAD))


def _lin_body(x_ref, w_ref, b_ref, o_ref):
  o_ref[...] = lax.dot_general(
      x_ref[...], w_ref[...], (((1,), (1,)), ((), ())),
      preferred_element_type=jnp.float32) + b_ref[...]


def _input_transform(x, w, b):
  return pl.pallas_call(
      _lin_body,
      grid=(NB,),
      in_specs=[
          pl.BlockSpec((ROWBLK, D), lambda i: (i, 0)),
          pl.BlockSpec((D, D), lambda i: (0, 0)),
          pl.BlockSpec((1, D), lambda i: (0, 0)),
      ],
      out_specs=pl.BlockSpec((ROWBLK, D), lambda i: (i, 0)),
      out_shape=jax.ShapeDtypeStruct((N, D), jnp.float32),
  )(x, w, b.reshape(1, D))


def _post_body(hd_ref, a0_ref, a1_ref, c_ref,
               wl_ref, bl_ref, wr_ref, g_ref, be_ref, o_ref):
  hd = hd_ref[...]
  agg = a0_ref[0] + a1_ref[0]
  cnt = jnp.sum(c_ref[...], axis=1, keepdims=True)
  mean = agg / jnp.maximum(cnt, 1.0)
  out = (lax.dot_general(mean, wl_ref[...], (((1,), (1,)), ((), ())),
                         preferred_element_type=jnp.float32)
         + bl_ref[...]
         + lax.dot_general(hd, wr_ref[...], (((1,), (1,)), ((), ())),
                           preferred_element_type=jnp.float32))
  nrm = jnp.sqrt(jnp.sum(out * out, axis=-1, keepdims=True))
  conv = out / jnp.maximum(nrm, 1e-12)
  y = hd + conv
  mu = jnp.mean(y, axis=-1, keepdims=True)
  var = jnp.mean((y - mu) ** 2, axis=-1, keepdims=True)
  o_ref[...] = (y - mu) / jnp.sqrt(var + 1e-5) * g_ref[...] + be_ref[...]


def _post(hd, agg_pair, cnt_hists, wl, bl, wr, g, be):
  return pl.pallas_call(
      _post_body,
      grid=(NB,),
      in_specs=[
          pl.BlockSpec((ROWBLK, D), lambda i: (i, 0)),
          pl.BlockSpec((1, ROWBLK, D), lambda i: (0, i, 0)),
          pl.BlockSpec((1, ROWBLK, D), lambda i: (1, i, 0)),
          pl.BlockSpec((ROWBLK, NW), lambda i: (i, 0)),
          pl.BlockSpec((D, D), lambda i: (0, 0)),
          pl.BlockSpec((1, D), lambda i: (0, 0)),
          pl.BlockSpec((D, D), lambda i: (0, 0)),
          pl.BlockSpec((1, D), lambda i: (0, 0)),
          pl.BlockSpec((1, D), lambda i: (0, 0)),
      ],
      out_specs=pl.BlockSpec((ROWBLK, D), lambda i: (i, 0)),
      out_shape=jax.ShapeDtypeStruct((N, D), jnp.float32),
  )(hd, agg_pair, agg_pair, cnt_hists, wl, bl.reshape(1, D), wr,
    g.reshape(1, D), be.reshape(1, D))


def kernel(x_user, x_item, edge_index_user_item, edge_index_item_user,
           W_user, b_user, W_item, b_item,
           Wl_ui, bl_ui, Wr_ui, Wl_iu, bl_iu, Wr_iu,
           ln_g_user, ln_b_user, ln_g_item, ln_b_item):
  h_user = _input_transform(x_user, W_user, b_user)
  h_item = _input_transform(x_item, W_item, b_item)

  agg, cnt = _sc_segment_sums(
      h_user, h_item,
      edge_index_user_item[0], edge_index_user_item[1],
      edge_index_item_user[0], edge_index_item_user[1])

  cnt_t = jnp.transpose(cnt, (0, 2, 1))
  out_item = _post(h_item, agg[0], cnt_t[0], Wl_ui, bl_ui, Wr_ui,
                   ln_g_item, ln_b_item)
  out_user = _post(h_user, agg[1], cnt_t[1], Wl_iu, bl_iu, Wr_iu,
                   ln_g_user, ln_b_user)
  return (out_user, out_item)
